# Initial kernel scaffold; baseline (speedup 1.0000x reference)
import jax
import jax.numpy as jnp
from jax.experimental import pallas as pl


def kernel(x):
    raise NotImplementedError("write your pallas kernel here")


# speedup vs baseline: 91.2065x; 91.2065x over previous
import math

import jax
import jax.numpy as jnp
from jax.experimental import pallas as pl

_INT_MIN = -2147483648


def _skey(f):
    i = jax.lax.bitcast_convert_type(f, jnp.int32)
    return jnp.where(i >= 0, i, jnp.int32(_INT_MIN) - i)


def _kth_largest_key(skey, k, count_fn, row_shape):
    K = jnp.full(row_shape, _INT_MIN, dtype=jnp.int32)
    for b in range(31, -1, -1):
        addend = _INT_MIN if b == 31 else (1 << b)
        cand = K + jnp.int32(addend)
        cnt = count_fn(skey >= _bcast_row(cand, skey.shape))
        K = jnp.where(cnt >= k, cand, K)
    return K


def _bcast_row(row_val, full_shape):
    return jnp.broadcast_to(row_val[..., None], full_shape)


def _make_kernel(k1, k2):
    def _kern(x_ref, o_ref):
        x = x_ref[...]
        s1 = _skey(x)

        def count1(ge):
            return jnp.sum(ge.astype(jnp.int32), axis=2)

        K1 = _kth_largest_key(s1, k1, count1, s1.shape[:2])
        keep1 = s1 >= _bcast_row(K1, s1.shape)
        m1 = jnp.where(keep1, x, 0.0)
        s2 = jnp.where(keep1, s1, 0)

        def count2(ge):
            c = jnp.sum(ge.astype(jnp.int32), axis=2)
            return jnp.sum(c, axis=0, keepdims=True)

        K2 = _kth_largest_key(s2, k2, count2, (1, s1.shape[1]))
        K2 = jnp.broadcast_to(K2, s1.shape[:2])
        o_ref[...] = jnp.where(s2 >= _bcast_row(K2, s1.shape), m1, 0.0)

    return _kern


def kernel(x):
    B, C, H, W = x.shape
    HW = H * W
    k1 = math.ceil(0.5 * H * W)
    k2 = math.ceil(0.5 * k1 * B)
    cb = C
    for c_try in (64, 32, 16, 8, 4, 2, 1):
        if C % c_try == 0:
            cb = c_try
            break
    xr = x.reshape(B, C, HW)
    out = pl.pallas_call(
        _make_kernel(k1, k2),
        grid=(C // cb,),
        in_specs=[pl.BlockSpec((B, cb, HW), lambda i: (0, i, 0))],
        out_specs=pl.BlockSpec((B, cb, HW), lambda i: (0, i, 0)),
        out_shape=jax.ShapeDtypeStruct((B, C, HW), jnp.float32),
    )(xr)
    return out.reshape(B, C, H, W)
